# Initial kernel scaffold; baseline (speedup 1.0000x reference)
import jax
import jax.numpy as jnp
from jax.experimental import pallas as pl


def kernel(input):
    raise NotImplementedError("write your pallas kernel here")


# speedup vs baseline: 15.0665x; 15.0665x over previous
import functools

import jax
import jax.numpy as jnp
from jax import lax
from jax.experimental import pallas as pl
from jax.experimental.pallas import tpu as pltpu
from jax.experimental.pallas import tpu_sc as plsc

L = 16
D = 4096
VPR = D // L
NROWS = 32 * 16 * 16
NC, NS = 2, 16
NW = NC * NS
ROWS_PER_W = NROWS // NW
RB = 8
NT = ROWS_PER_W // RB
BISECT = 14


def _recip(x):
    xi = lax.bitcast_convert_type(x, jnp.int32)
    r = lax.bitcast_convert_type(jnp.int32(0x7EF311C3) - xi, jnp.float32)
    for _ in range(3):
        r = r * (2.0 - x * r)
    return r


def _row_sparsemax(zbuf, obuf, cbuf, r):
    def mx_body(i, acc):
        return jnp.maximum(acc, zbuf[r, pl.ds(i * L, L)])
    accm = lax.fori_loop(0, VPR, mx_body, jnp.full((L,), -jnp.inf, jnp.float32))
    mx = jnp.max(accm)
    lo0 = mx - 1.0

    def cp_body(i, cnt):
        v = zbuf[r, pl.ds(i * L, L)]
        m = v > lo0
        plsc.store_compressed(cbuf.at[pl.ds(cnt, L)], v, mask=m)
        return cnt + jnp.sum(m.astype(jnp.int32))
    cnt = lax.fori_loop(0, VPR, cp_body, jnp.int32(0))
    cbuf[pl.ds(cnt, L)] = jnp.full((L,), lo0 - 1.0, jnp.float32)
    nv = lax.shift_right_logical(cnt + (L - 1), 4)

    def bis_body(_, lohi):
        lo, hi = lohi
        t = 0.5 * (lo + hi)
        def f_body(i, acc):
            v = cbuf[pl.ds(i * L, L)]
            return acc + jnp.maximum(v - t, 0.0)
        acc = lax.fori_loop(0, nv, f_body, jnp.zeros((L,), jnp.float32))
        ge = (jnp.sum(acc) - 1.0) >= 0.0
        return jnp.where(ge, t, lo), jnp.where(ge, hi, t)
    lo, _hi = lax.fori_loop(0, BISECT, bis_body, (lo0, mx))

    def nf_body(i, acc):
        v = cbuf[pl.ds(i * L, L)]
        m = v > lo
        sacc, kacc = acc
        return (sacc + jnp.where(m, v, 0.0), kacc + jnp.where(m, 1.0, 0.0))
    sacc, kacc = lax.fori_loop(
        0, nv, nf_body,
        (jnp.zeros((L,), jnp.float32), jnp.zeros((L,), jnp.float32)))
    tau = (jnp.sum(sacc) - 1.0) * _recip(jnp.maximum(jnp.sum(kacc), 1.0))

    def out_body(i, carry):
        v = zbuf[r, pl.ds(i * L, L)]
        obuf[r, pl.ds(i * L, L)] = jnp.maximum(v - tau, 0.0)
        return carry
    lax.fori_loop(0, VPR, out_body, 0)


def _sc_body(x_hbm, out_hbm, zbuf, obuf, cbuf):
    wid = lax.axis_index("s") * NC + lax.axis_index("c")
    row0 = wid * ROWS_PER_W

    def block_body(tidx, carry):
        base = row0 + tidx * RB
        pltpu.sync_copy(x_hbm.at[pl.ds(base, RB)], zbuf)
        def row_body(r, c):
            _row_sparsemax(zbuf, obuf, cbuf, r)
            return c
        lax.fori_loop(0, RB, row_body, 0)
        pltpu.sync_copy(obuf, out_hbm.at[pl.ds(base, RB)])
        return carry
    lax.fori_loop(0, NT, block_body, 0)


@jax.jit
def kernel(input):
    x = input.reshape(NROWS, D)
    mesh = plsc.VectorSubcoreMesh(
        core_axis_name="c", subcore_axis_name="s", num_cores=NC,
        num_subcores=NS)
    run = functools.partial(
        pl.kernel,
        out_type=jax.ShapeDtypeStruct((NROWS, D), jnp.float32),
        mesh=mesh,
        compiler_params=pltpu.CompilerParams(needs_layout_passes=False),
        scratch_types=[
            pltpu.VMEM((RB, D), jnp.float32),
            pltpu.VMEM((RB, D), jnp.float32),
            pltpu.VMEM((D + L,), jnp.float32),
        ],
    )(_sc_body)
    return run(x).reshape(input.shape)


# speedup vs baseline: 16.3677x; 1.0864x over previous
import functools

import jax
import jax.numpy as jnp
from jax import lax
from jax.experimental import pallas as pl
from jax.experimental.pallas import tpu as pltpu
from jax.experimental.pallas import tpu_sc as plsc

L = 16
D = 4096
VPR = D // L
NROWS = 32 * 16 * 16
NC, NS = 2, 16
NW = NC * NS
ROWS_PER_W = NROWS // NW
RB = 8
NT = ROWS_PER_W // RB
BISECT = 14


def _recip(x):
    xi = lax.bitcast_convert_type(x, jnp.int32)
    r = lax.bitcast_convert_type(jnp.int32(0x7EF311C3) - xi, jnp.float32)
    for _ in range(3):
        r = r * (2.0 - x * r)
    return r


UMAX = 8
UCP = 4
UOUT = 8


def _row_sparsemax(zbuf, obuf, cbuf, r):
    def mx_body(i, accs):
        base = i * (L * UMAX)
        return tuple(
            jnp.maximum(a, zbuf[r, pl.ds(base + j * L, L)])
            for j, a in enumerate(accs))
    accs = lax.fori_loop(
        0, VPR // UMAX, mx_body,
        (jnp.full((L,), -jnp.inf, jnp.float32),) * UMAX)
    accm = functools.reduce(jnp.maximum, accs)
    mx = jnp.max(accm)
    lo0 = mx - 1.0

    def cp_body(i, cnt):
        base = i * (L * UCP)
        for j in range(UCP):
            v = zbuf[r, pl.ds(base + j * L, L)]
            m = v > lo0
            plsc.store_compressed(cbuf.at[pl.ds(cnt, L)], v, mask=m)
            cnt = cnt + plsc.all_reduce_population_count(m)[0]
        return cnt
    cnt = lax.fori_loop(0, VPR // UCP, cp_body, jnp.int32(0))
    cbuf[pl.ds(cnt, L)] = jnp.full((L,), lo0 - 1.0, jnp.float32)
    nv = lax.shift_right_logical(cnt + (L - 1), 4)

    def bis_body(_, lohi):
        lo, hi = lohi
        t = 0.5 * (lo + hi)
        def f_body(i, acc):
            v = cbuf[pl.ds(i * L, L)]
            return acc + jnp.maximum(v - t, 0.0)
        acc = lax.fori_loop(0, nv, f_body, jnp.zeros((L,), jnp.float32))
        ge = (jnp.sum(acc) - 1.0) >= 0.0
        return jnp.where(ge, t, lo), jnp.where(ge, hi, t)
    lo, _hi = lax.fori_loop(0, BISECT, bis_body, (lo0, mx))

    def nf_body(i, acc):
        v = cbuf[pl.ds(i * L, L)]
        m = v > lo
        sacc, kacc = acc
        return (sacc + jnp.where(m, v, 0.0), kacc + jnp.where(m, 1.0, 0.0))
    sacc, kacc = lax.fori_loop(
        0, nv, nf_body,
        (jnp.zeros((L,), jnp.float32), jnp.zeros((L,), jnp.float32)))
    tau = (jnp.sum(sacc) - 1.0) * _recip(jnp.maximum(jnp.sum(kacc), 1.0))

    def out_body(i, carry):
        base = i * (L * UOUT)
        for j in range(UOUT):
            v = zbuf[r, pl.ds(base + j * L, L)]
            obuf[r, pl.ds(base + j * L, L)] = jnp.maximum(v - tau, 0.0)
        return carry
    lax.fori_loop(0, VPR // UOUT, out_body, 0)


def _sc_body(x_hbm, out_hbm, zbuf, obuf, cbuf):
    wid = lax.axis_index("s") * NC + lax.axis_index("c")
    row0 = wid * ROWS_PER_W

    def block_body(tidx, carry):
        base = row0 + tidx * RB
        pltpu.sync_copy(x_hbm.at[pl.ds(base, RB)], zbuf)
        def row_body(r, c):
            _row_sparsemax(zbuf, obuf, cbuf, r)
            return c
        lax.fori_loop(0, RB, row_body, 0)
        pltpu.sync_copy(obuf, out_hbm.at[pl.ds(base, RB)])
        return carry
    lax.fori_loop(0, NT, block_body, 0)


@jax.jit
def kernel(input):
    x = input.reshape(NROWS, D)
    mesh = plsc.VectorSubcoreMesh(
        core_axis_name="c", subcore_axis_name="s", num_cores=NC,
        num_subcores=NS)
    run = functools.partial(
        pl.kernel,
        out_type=jax.ShapeDtypeStruct((NROWS, D), jnp.float32),
        mesh=mesh,
        compiler_params=pltpu.CompilerParams(needs_layout_passes=False),
        scratch_types=[
            pltpu.VMEM((RB, D), jnp.float32),
            pltpu.VMEM((RB, D), jnp.float32),
            pltpu.VMEM((D + L,), jnp.float32),
        ],
    )(_sc_body)
    return run(x).reshape(input.shape)
